# Initial kernel scaffold; baseline (speedup 1.0000x reference)
import jax
import jax.numpy as jnp
from jax.experimental import pallas as pl


def kernel(x, edge_attr, W_e, b_e, W1, b1, W2, b2, gamma, beta, p_pool, W_g, b_g, edge_index, batch):
    raise NotImplementedError("write your pallas kernel here")


# speedup vs baseline: 3.6259x; 3.6259x over previous
import functools
import math

import jax
import jax.numpy as jnp
from jax import lax
from jax.experimental import pallas as pl
from jax.experimental.pallas import tpu as pltpu
from jax.experimental.pallas import tpu_sc as plsc

N = 10000
E = 160000
D = 256
EDIM = 16
G = 40
NPER = 250
L = 3
RATIO = 0.8

H = 128
NC = 2
NS = 16
EPS = E // NS
CH = 80
NCHUNK = EPS // CH
RPS = N // NS
NEG = -1e30

NB = 2000
NBLK = N // NB


def _topk_sizes():
    ks = []
    c = NPER
    for _ in range(L):
        c = math.ceil(RATIO * c)
        ks.append(c)
    return ks


_KS = _topk_sizes()
_CNT = [float(G * NPER)] + [float(G * k) for k in _KS[:-1]]


def _mp_body(x2, lin2, src, dst, aggr, srcv, dstv, xbuf, lbuf, aggr_sh, sem):
    c = lax.axis_index("c")
    s = lax.axis_index("s")

    def _zb(r, carry):
        for l in range(H // 16):
            xbuf[r, pl.ds(l * 16, 16)] = jnp.zeros((16,), jnp.float32)
        return carry

    lax.fori_loop(0, CH, _zb, 0)
    row0 = s * 624 + 8 * jnp.minimum(s, 2)

    def _zero_range(nrows):
        nfull = nrows // CH
        rem = nrows - nfull * CH
        for jj in range(nfull):
            pltpu.sync_copy(xbuf, aggr_sh.at[pl.ds(row0 + jj * CH, CH)])
        if rem:
            pltpu.sync_copy(xbuf.at[pl.ds(0, rem)],
                            aggr_sh.at[pl.ds(row0 + nfull * CH, rem)])

    @pl.when(s < 2)
    def _():
        _zero_range(632)

    @pl.when(s >= 2)
    def _():
        _zero_range(624)

    plsc.subcore_barrier()

    def _chunk(j, carry):
        base = s * EPS + j * CH
        pltpu.sync_copy(src.at[pl.ds(base, CH)], srcv)
        pltpu.sync_copy(dst.at[pl.ds(base, CH)], dstv)
        coff = c * N
        for i in range(CH // 16):
            sl = pl.ds(i * 16, 16)
            srcv[sl] = srcv[sl] + coff
        gat = pltpu.async_copy(x2.at[srcv], xbuf, sem)
        pltpu.sync_copy(lin2.at[pl.ds(c * E + base, CH)], lbuf)
        gat.wait()

        def _cmp(r, cc):
            for l in range(H // 16):
                sl = pl.ds(l * 16, 16)
                xbuf[r, sl] = jnp.maximum(xbuf[r, sl] + lbuf[r, sl], 0.0)
            return cc

        lax.fori_loop(0, CH, _cmp, 0)
        pltpu.sync_copy(xbuf, aggr_sh.at[dstv], add=True)
        return carry

    lax.fori_loop(0, NCHUNK, _chunk, 0)
    plsc.subcore_barrier()

    @pl.when(s < 2)
    def _():
        pltpu.sync_copy(aggr_sh.at[pl.ds(row0, 632)],
                        aggr.at[c, pl.ds(row0, 632)])

    @pl.when(s >= 2)
    def _():
        pltpu.sync_copy(aggr_sh.at[pl.ds(row0, 624)],
                        aggr.at[c, pl.ds(row0, 624)])


def _mp_call(x2, lin2, src, dst):
    mesh = plsc.VectorSubcoreMesh(
        core_axis_name="c", subcore_axis_name="s", num_cores=NC, num_subcores=NS)
    f = pl.kernel(
        _mp_body,
        out_type=jax.ShapeDtypeStruct((NC, N, H), jnp.float32),
        mesh=mesh,
        scratch_types=[
            pltpu.VMEM((CH,), jnp.int32),
            pltpu.VMEM((CH,), jnp.int32),
            pltpu.VMEM((CH, H), jnp.float32),
            pltpu.VMEM((CH, H), jnp.float32),
            pltpu.VMEM_SHARED((N, H), jnp.float32),
            pltpu.SemaphoreType.DMA,
        ],
    )
    return f(x2, lin2, src, dst)


def _lin_body(ea, w, b, o):
    o[0, 0] = jnp.dot(ea[...], w[0], preferred_element_type=jnp.float32) + b[0]


def _lin_call(edge_attr, W_e, b_e):
    EB = 4000
    return pl.pallas_call(
        _lin_body,
        grid=(L, NC, E // EB),
        in_specs=[
            pl.BlockSpec((EB, EDIM), lambda l, c, e: (e, 0)),
            pl.BlockSpec((1, EDIM, H), lambda l, c, e: (l, 0, c)),
            pl.BlockSpec((1, 1, H), lambda l, c, e: (l, 0, c)),
        ],
        out_specs=pl.BlockSpec((1, 1, EB, H), lambda l, c, e: (l, c, e, 0)),
        out_shape=jax.ShapeDtypeStruct((L, NC, E, H), jnp.float32),
    )(edge_attr, W_e, b_e.reshape(L, 1, D))


def _tc1a_body(x, a0, a1, m, w1, b1, w2, b2, h2o, s1, s2):
    @pl.when(pl.program_id(0) == 0)
    def _():
        s1[...] = jnp.zeros_like(s1)
        s2[...] = jnp.zeros_like(s2)

    h0 = x[...] + jnp.concatenate([a0[...], a1[...]], axis=1)
    h1 = jnp.maximum(
        jnp.dot(h0, w1[...], preferred_element_type=jnp.float32) + b1[...], 0.0)
    h2 = (jnp.dot(h1, w2[...], preferred_element_type=jnp.float32) + b2[...]) * m[...]
    h2o[...] = h2
    s1[...] += jnp.sum(h2, axis=0, keepdims=True)
    s2[...] += jnp.sum(h2 * h2, axis=0, keepdims=True)


def _tc1a_call(x, a0, a1, m, w1, b1, w2, b2):
    return pl.pallas_call(
        _tc1a_body,
        grid=(NBLK,),
        in_specs=[
            pl.BlockSpec((NB, D), lambda i: (i, 0)),
            pl.BlockSpec((NB, H), lambda i: (i, 0)),
            pl.BlockSpec((NB, H), lambda i: (i, 0)),
            pl.BlockSpec((NB, 1), lambda i: (i, 0)),
            pl.BlockSpec((D, D), lambda i: (0, 0)),
            pl.BlockSpec((1, D), lambda i: (0, 0)),
            pl.BlockSpec((D, D), lambda i: (0, 0)),
            pl.BlockSpec((1, D), lambda i: (0, 0)),
        ],
        out_specs=[
            pl.BlockSpec((NB, D), lambda i: (i, 0)),
            pl.BlockSpec((1, D), lambda i: (0, 0)),
            pl.BlockSpec((1, D), lambda i: (0, 0)),
        ],
        out_shape=[
            jax.ShapeDtypeStruct((N, D), jnp.float32),
            jax.ShapeDtypeStruct((1, D), jnp.float32),
            jax.ShapeDtypeStruct((1, D), jnp.float32),
        ],
    )(x, a0, a1, m, w1, b1, w2, b2)


def _tc1b_body(h2, m, s1, s2, g, be, p, wg, h4o, sco, qo, *, cnt):
    mean = s1[...] / cnt
    var = s2[...] / cnt - mean * mean
    h3 = (h2[...] - mean) / jnp.sqrt(var + 1e-5) * g[...] + be[...]
    h4 = jnp.maximum(h3, 0.0) * m[...]
    h4o[...] = h4
    pv = p[...]
    nrm = jnp.sqrt(jnp.sum(pv * pv))
    sco[...] = jnp.dot(h4, pv, preferred_element_type=jnp.float32) / nrm
    qo[...] = jnp.dot(h4, wg[...], preferred_element_type=jnp.float32)


def _tc1b_call(h2, m, s1, s2, g, be, p, wg, cnt):
    return pl.pallas_call(
        functools.partial(_tc1b_body, cnt=cnt),
        grid=(NBLK,),
        in_specs=[
            pl.BlockSpec((NB, D), lambda i: (i, 0)),
            pl.BlockSpec((NB, 1), lambda i: (i, 0)),
            pl.BlockSpec((1, D), lambda i: (0, 0)),
            pl.BlockSpec((1, D), lambda i: (0, 0)),
            pl.BlockSpec((1, D), lambda i: (0, 0)),
            pl.BlockSpec((1, D), lambda i: (0, 0)),
            pl.BlockSpec((D, 1), lambda i: (0, 0)),
            pl.BlockSpec((D, 1), lambda i: (0, 0)),
        ],
        out_specs=[
            pl.BlockSpec((NB, D), lambda i: (i, 0)),
            pl.BlockSpec((NB, 1), lambda i: (i, 0)),
            pl.BlockSpec((NB, 1), lambda i: (i, 0)),
        ],
        out_shape=[
            jax.ShapeDtypeStruct((N, D), jnp.float32),
            jax.ShapeDtypeStruct((N, 1), jnp.float32),
            jax.ShapeDtypeStruct((N, 1), jnp.float32),
        ],
    )(h2, m, s1, s2, g, be, p, wg)


def _tc2_body(s_ref, m_ref, q_ref, bg_ref, nmo, sco, cfo, *, k):
    s = s_ref[...]
    m = m_ref[...]
    sm = jnp.where(m > 0, s, -jnp.inf)
    i32 = lax.bitcast_convert_type(sm, jnp.int32)
    key = jnp.where(i32 < 0, i32 ^ jnp.int32(0x7FFFFFFF), i32)
    t = jnp.full((G, 1), jnp.int32(-2**31))
    for bit in range(31, -1, -1):
        if bit == 31:
            cand = jnp.zeros((G, 1), jnp.int32)
        else:
            cand = t | jnp.int32(1 << bit)
        cnt_ge = jnp.sum(jnp.where(key >= cand, 1.0, 0.0), axis=1, keepdims=True)
        t = jnp.where(cnt_ge >= float(k), cand, t)
    gt = (key > t).astype(jnp.float32)
    eq = (key == t).astype(jnp.float32)
    need = float(k) - jnp.sum(gt, axis=1, keepdims=True)
    ia = lax.broadcasted_iota(jnp.int32, (NPER, NPER), 0)
    ib = lax.broadcasted_iota(jnp.int32, (NPER, NPER), 1)
    ut = (ia < ib).astype(jnp.float32)
    prefix = jnp.dot(eq, ut, preferred_element_type=jnp.float32)
    newm = gt + eq * (prefix < need).astype(jnp.float32)
    nmo[...] = newm
    scale = jnp.tanh(s) * newm
    sco[...] = scale
    gate = scale * q_ref[...] + bg_ref[...]
    gate = jnp.where(newm > 0, gate, -1e30)
    gmax = jnp.max(gate, axis=1, keepdims=True)
    ew = jnp.exp(gate - gmax) * newm
    alpha = ew / jnp.sum(ew, axis=1, keepdims=True)
    cfo[...] = alpha * scale


def _tc2_call(s_gp, m_gp, q_gp, bg, k):
    return pl.pallas_call(
        functools.partial(_tc2_body, k=k),
        out_shape=[
            jax.ShapeDtypeStruct((G, NPER), jnp.float32),
            jax.ShapeDtypeStruct((G, NPER), jnp.float32),
            jax.ShapeDtypeStruct((G, NPER), jnp.float32),
        ],
    )(s_gp, m_gp, q_gp, bg)


def _indicator():
    ig = lax.broadcasted_iota(jnp.int32, (G, N), 0)
    inn = lax.broadcasted_iota(jnp.int32, (G, N), 1)
    return (inn // NPER == ig).astype(jnp.float32)


def _tc3_body(h4, scale, coef, nm, outp, xno, x2o, outno):
    h = h4[...]
    xn = h * scale[...]
    xno[...] = xn
    alive = nm[...] > 0
    x2o[0:N, :] = jnp.where(alive, xn[:, 0:H], NEG)
    x2o[N:2 * N, :] = jnp.where(alive, xn[:, H:D], NEG)
    ind = _indicator()
    outno[...] = outp[...] + jnp.dot(ind, coef[...] * h,
                                     preferred_element_type=jnp.float32,
                                     precision=lax.Precision.HIGHEST)


def _tc3_call(h4, scale, coef, nm, outp):
    return pl.pallas_call(
        _tc3_body,
        out_shape=[
            jax.ShapeDtypeStruct((N, D), jnp.float32),
            jax.ShapeDtypeStruct((2 * N, H), jnp.float32),
            jax.ShapeDtypeStruct((G, D), jnp.float32),
        ],
    )(h4, scale, coef, nm, outp)


def _tc3f_body(h4, coef, outp, outno):
    ind = _indicator()
    outno[...] = outp[...] + jnp.dot(ind, coef[...] * h4[...],
                                     preferred_element_type=jnp.float32,
                                     precision=lax.Precision.HIGHEST)


def _tc3f_call(h4, coef, outp):
    return pl.pallas_call(
        _tc3f_body,
        out_shape=jax.ShapeDtypeStruct((G, D), jnp.float32),
    )(h4, coef, outp)


def kernel(x, edge_attr, W_e, b_e, W1, b1, W2, b2, gamma, beta, p_pool, W_g,
           b_g, edge_index, batch):
    src = edge_index[0]
    dst = edge_index[1]
    lin_all = _lin_call(edge_attr, W_e, b_e)
    x2 = jnp.concatenate([x[:, 0:H], x[:, H:D]], axis=0)
    xcur = x
    mask_col = jnp.ones((N, 1), jnp.float32)
    out = jnp.zeros((G, D), jnp.float32)
    bg = b_g.reshape(1, 1)
    for i in range(L):
        lin2 = lin_all[i].reshape(NC * E, H)
        aggr = _mp_call(x2, lin2, src, dst)
        h2, s1, s2 = _tc1a_call(
            xcur, aggr[0], aggr[1], mask_col,
            W1[i], b1[i].reshape(1, D), W2[i], b2[i].reshape(1, D))
        h4, score_col, q_col = _tc1b_call(
            h2, mask_col, s1, s2,
            gamma[i].reshape(1, D), beta[i].reshape(1, D),
            p_pool[i].reshape(D, 1), W_g, _CNT[i])
        newm_gp, scale_gp, coef_gp = _tc2_call(
            score_col.reshape(G, NPER), mask_col.reshape(G, NPER),
            q_col.reshape(G, NPER), bg, _KS[i])
        scale_col = scale_gp.reshape(N, 1)
        coef_col = coef_gp.reshape(N, 1)
        if i < L - 1:
            xcur, x2, out = _tc3_call(
                h4, scale_col, coef_col, newm_gp.reshape(N, 1), out)
        else:
            out = _tc3f_call(h4, coef_col, out)
        mask_col = newm_gp.reshape(N, 1)
    return out
